# baseline (speedup 1.0000x reference)
import functools

import jax
import jax.numpy as jnp
from jax import lax
from jax.experimental import pallas as pl
from jax.experimental.pallas import tpu as pltpu
from jax.experimental.pallas import tpu_sc as plsc

_NUM_CORES = 2
_NUM_SUBCORES = 16
_NW = _NUM_CORES * _NUM_SUBCORES
_LANES = 16


@functools.lru_cache(maxsize=None)
def _build(B, T, DIM):
    total = T * DIM
    per_w = total // _NW
    C = 16384
    if per_w % C:
        C = per_w
    n_sub = per_w // C

    mesh = plsc.VectorSubcoreMesh(core_axis_name="c", subcore_axis_name="s")

    @functools.partial(
        pl.kernel,
        out_type=jax.ShapeDtypeStruct((B, total), jnp.float32),
        mesh=mesh,
        scratch_types=[
            pltpu.VMEM((C,), jnp.float32),
            pltpu.VMEM((C,), jnp.float32),
        ],
    )
    def k(x_hbm, pos_hbm, out_hbm, pos_v, x_v):
        wid = lax.axis_index("s") * _NUM_CORES + lax.axis_index("c")
        base = wid * per_w
        for s in range(n_sub):
            off = base + s * C
            pltpu.sync_copy(pos_hbm.at[pl.ds(off, C)], pos_v)
            for b in range(B):
                pltpu.sync_copy(x_hbm.at[b, pl.ds(off, C)], x_v)

                def body(i, _):
                    sl = pl.ds(i * _LANES, _LANES)
                    x_v[sl] = x_v[sl] + pos_v[sl]
                    return _

                lax.fori_loop(0, C // _LANES, body, 0)
                pltpu.sync_copy(x_v, out_hbm.at[b, pl.ds(off, C)])

    return k


def kernel(x, pos):
    B, T, DIM = x.shape
    x_flat = x.reshape(B, T * DIM)
    pos_flat = pos[:T].reshape(T * DIM)
    out = _build(B, T, DIM)(x_flat, pos_flat)
    return out.reshape(B, T, DIM)


# speedup vs baseline: 1.8612x; 1.8612x over previous
import functools

import jax
import jax.numpy as jnp
from jax import lax
from jax.experimental import pallas as pl
from jax.experimental.pallas import tpu as pltpu
from jax.experimental.pallas import tpu_sc as plsc

_NUM_CORES = 2
_NUM_SUBCORES = 16
_NW = _NUM_CORES * _NUM_SUBCORES
_LANES = 16
_C = 16384


@functools.lru_cache(maxsize=None)
def _build(B, T, DIM):
    total = T * DIM
    per_w = total // _NW
    C = _C if per_w % _C == 0 else per_w
    n_sub = per_w // C
    n_tiles = n_sub * B

    mesh = plsc.VectorSubcoreMesh(core_axis_name="c", subcore_axis_name="s")

    @functools.partial(
        pl.kernel,
        out_type=jax.ShapeDtypeStruct((B, total), jnp.float32),
        mesh=mesh,
        scratch_types=(
            [pltpu.VMEM((C,), jnp.float32) for _ in range(3)]
            + [pltpu.VMEM((C,), jnp.float32) for _ in range(2)]
            + [pltpu.SemaphoreType.DMA for _ in range(8)]
        ),
    )
    def k(x_hbm, pos_hbm, out_hbm,
          xv0, xv1, xv2, pv0, pv1,
          sxi0, sxi1, sxi2, soo0, soo1, soo2, spi0, spi1):
        wid = lax.axis_index("s") * _NUM_CORES + lax.axis_index("c")
        base = wid * per_w
        xv = (xv0, xv1, xv2)
        pv = (pv0, pv1)
        sxi = (sxi0, sxi1, sxi2)
        soo = (soo0, soo1, soo2)
        spi = (spi0, spi1)

        def x_loc(kk):
            s, b = divmod(kk, B)
            return b, base + s * C

        def start_xin(kk):
            b, off = x_loc(kk)
            return pltpu.async_copy(
                x_hbm.at[b, pl.ds(off, C)], xv[kk % 3], sxi[kk % 3])

        def start_pin(s):
            return pltpu.async_copy(
                pos_hbm.at[pl.ds(base + s * C, C)], pv[s % 2], spi[s % 2])

        def start_out(kk):
            b, off = x_loc(kk)
            return pltpu.async_copy(
                xv[kk % 3], out_hbm.at[b, pl.ds(off, C)], soo[kk % 3])

        pending = {}
        pending["p0"] = start_pin(0)
        for j in range(min(3, n_tiles)):
            pending[f"x{j}"] = start_xin(j)

        for kk in range(n_tiles):
            s, b = divmod(kk, B)
            if b == 0:
                pending.pop(f"p{s}").wait()
                if s + 1 < n_sub:
                    pending[f"p{s + 1}"] = start_pin(s + 1)
            pending.pop(f"x{kk}").wait()

            xbuf = xv[kk % 3]
            pbuf = pv[s % 2]

            @plsc.parallel_loop(0, C // _LANES, step=1, unroll=8)
            def add_body(i):
                sl = pl.ds(i * _LANES, _LANES)
                plsc.addupdate(xbuf.at[sl], pbuf[sl])

            pending[f"o{kk}"] = start_out(kk)
            if kk >= 1 and kk + 2 < n_tiles:
                pending.pop(f"o{kk - 1}").wait()
                pending[f"x{kk + 2}"] = start_xin(kk + 2)

        for h in pending.values():
            h.wait()

    return k


def kernel(x, pos):
    B, T, DIM = x.shape
    x_flat = x.reshape(B, T * DIM)
    pos_flat = pos[:T].reshape(T * DIM)
    out = _build(B, T, DIM)(x_flat, pos_flat)
    return out.reshape(B, T, DIM)


# speedup vs baseline: 4.5728x; 2.4568x over previous
import functools

import jax
import jax.numpy as jnp
from jax import lax
from jax.experimental import pallas as pl
from jax.experimental.pallas import tpu as pltpu
from jax.experimental.pallas import tpu_sc as plsc

_NUM_CORES = 2
_NUM_SUBCORES = 16
_NW = _NUM_CORES * _NUM_SUBCORES
_LANES = 16
_R = 16


@functools.lru_cache(maxsize=None)
def _build(B, T, DIM):
    rows_w = T // _NW
    R = _R if rows_w % _R == 0 else rows_w
    n_sub = rows_w // R
    n_tiles = n_sub * B
    groups_row = DIM // _LANES

    mesh = plsc.VectorSubcoreMesh(core_axis_name="c", subcore_axis_name="s")

    @functools.partial(
        pl.kernel,
        out_type=jax.ShapeDtypeStruct((B, T, DIM), jnp.float32),
        mesh=mesh,
        compiler_params=pltpu.CompilerParams(use_tc_tiling_on_sc=True),
        scratch_types=(
            [pltpu.VMEM((R, DIM), jnp.float32) for _ in range(3)]
            + [pltpu.VMEM((R, DIM), jnp.float32) for _ in range(2)]
            + [pltpu.SemaphoreType.DMA for _ in range(8)]
        ),
    )
    def k(x_hbm, pos_hbm, out_hbm,
          xv0, xv1, xv2, pv0, pv1,
          sxi0, sxi1, sxi2, soo0, soo1, soo2, spi0, spi1):
        wid = lax.axis_index("s") * _NUM_CORES + lax.axis_index("c")
        base = wid * rows_w
        xv = (xv0, xv1, xv2)
        pv = (pv0, pv1)
        sxi = (sxi0, sxi1, sxi2)
        soo = (soo0, soo1, soo2)
        spi = (spi0, spi1)

        def x_loc(kk):
            s, b = divmod(kk, B)
            return b, base + s * R

        def start_xin(kk):
            b, r0 = x_loc(kk)
            return pltpu.async_copy(
                x_hbm.at[b, pl.ds(r0, R), :], xv[kk % 3], sxi[kk % 3])

        def start_pin(s):
            return pltpu.async_copy(
                pos_hbm.at[pl.ds(base + s * R, R), :], pv[s % 2], spi[s % 2])

        def start_out(kk):
            b, r0 = x_loc(kk)
            return pltpu.async_copy(
                xv[kk % 3], out_hbm.at[b, pl.ds(r0, R), :], soo[kk % 3])

        pending = {}
        pending["p0"] = start_pin(0)
        for j in range(min(3, n_tiles)):
            pending[f"x{j}"] = start_xin(j)

        for kk in range(n_tiles):
            s, b = divmod(kk, B)
            if b == 0:
                pending.pop(f"p{s}").wait()
                if s + 1 < n_sub:
                    pending[f"p{s + 1}"] = start_pin(s + 1)
            pending.pop(f"x{kk}").wait()

            xbuf = xv[kk % 3]
            pbuf = pv[s % 2]

            @plsc.parallel_loop(0, R * groups_row, step=1, unroll=8)
            def add_body(i):
                r = i // groups_row
                sl = pl.ds((i % groups_row) * _LANES, _LANES)
                xbuf[r, sl] = xbuf[r, sl] + pbuf[r, sl]

            pending[f"o{kk}"] = start_out(kk)
            if kk >= 1 and kk + 2 < n_tiles:
                pending.pop(f"o{kk - 1}").wait()
                pending[f"x{kk + 2}"] = start_xin(kk + 2)

        for h in pending.values():
            h.wait()

    return k


def kernel(x, pos):
    B, T, DIM = x.shape
    return _build(B, T, DIM)(x, pos[:T])


# speedup vs baseline: 4.5928x; 1.0044x over previous
import functools

import jax
import jax.numpy as jnp
from jax import lax
from jax.experimental import pallas as pl
from jax.experimental.pallas import tpu as pltpu
from jax.experimental.pallas import tpu_sc as plsc

_NUM_CORES = 2
_NUM_SUBCORES = 16
_NW = _NUM_CORES * _NUM_SUBCORES
_LANES = 16
_R = 16


@functools.lru_cache(maxsize=None)
def _build(B, T, DIM):
    rows_w = T // _NW
    R = _R if rows_w % _R == 0 else rows_w
    n_sub = rows_w // R
    n_tiles = n_sub * B
    groups_row = DIM // _LANES

    mesh = plsc.VectorSubcoreMesh(core_axis_name="c", subcore_axis_name="s")

    @functools.partial(
        pl.kernel,
        out_type=jax.ShapeDtypeStruct((B, T, DIM), jnp.float32),
        mesh=mesh,
        compiler_params=pltpu.CompilerParams(use_tc_tiling_on_sc=True),
        scratch_types=(
            [pltpu.VMEM((R, DIM), jnp.float32) for _ in range(4)]
            + [pltpu.VMEM((R, DIM), jnp.float32) for _ in range(2)]
            + [pltpu.SemaphoreType.DMA for _ in range(10)]
        ),
    )
    def k(x_hbm, pos_hbm, out_hbm,
          xv0, xv1, xv2, xv3, pv0, pv1,
          sxi0, sxi1, sxi2, sxi3, soo0, soo1, soo2, soo3, spi0, spi1):
        wid = lax.axis_index("s") * _NUM_CORES + lax.axis_index("c")
        base = wid * rows_w
        xv = (xv0, xv1, xv2, xv3)
        pv = (pv0, pv1)
        sxi = (sxi0, sxi1, sxi2, sxi3)
        soo = (soo0, soo1, soo2, soo3)
        spi = (spi0, spi1)

        def x_loc(kk):
            s, b = divmod(kk, B)
            return b, base + s * R

        def start_xin(kk):
            b, r0 = x_loc(kk)
            return pltpu.async_copy(
                x_hbm.at[b, pl.ds(r0, R), :], xv[kk % 4], sxi[kk % 4])

        def start_pin(s):
            return pltpu.async_copy(
                pos_hbm.at[pl.ds(base + s * R, R), :], pv[s % 2], spi[s % 2])

        def start_out(kk):
            b, r0 = x_loc(kk)
            return pltpu.async_copy(
                xv[kk % 4], out_hbm.at[b, pl.ds(r0, R), :], soo[kk % 4])

        pending = {}
        pending["p0"] = start_pin(0)
        for j in range(min(4, n_tiles)):
            pending[f"x{j}"] = start_xin(j)

        for kk in range(n_tiles):
            s, b = divmod(kk, B)
            if b == 0:
                pending.pop(f"p{s}").wait()
                if s + 1 < n_sub:
                    pending[f"p{s + 1}"] = start_pin(s + 1)
            pending.pop(f"x{kk}").wait()

            xbuf = xv[kk % 4]
            pbuf = pv[s % 2]

            @plsc.parallel_loop(0, R * groups_row, step=1, unroll=8)
            def add_body(i):
                r = i // groups_row
                sl = pl.ds((i % groups_row) * _LANES, _LANES)
                xbuf[r, sl] = xbuf[r, sl] + pbuf[r, sl]

            pending[f"o{kk}"] = start_out(kk)
            if kk >= 2 and kk + 2 < n_tiles:
                pending.pop(f"o{kk - 2}").wait()
                pending[f"x{kk + 2}"] = start_xin(kk + 2)

        for h in pending.values():
            h.wait()

    return k


def kernel(x, pos):
    B, T, DIM = x.shape
    return _build(B, T, DIM)(x, pos[:T])
